# baseline (speedup 1.0000x reference)
import functools

import jax
import jax.numpy as jnp
from jax.experimental import pallas as pl
from jax.experimental.pallas import tpu as pltpu
from jax.experimental.pallas import tpu_sc as plsc

_D = 256
_K = 8192
_C = 32
_BETA = 0.25
_RB = 256
_N = 8 * 576
_NB = _N // _RB
_INT_MIN = -2147483648
_INT16_BIG = 8192

_NW = 32
_BPW = _N // _NW


def _rowsumsq(x):
    y = x * x
    s = y[:, :128] + y[:, 128:]
    p = s[:, 0:8] + s[:, 8:16]
    for g in range(2, 16):
        p = p + s[:, 8 * g:8 * g + 8]
    q = p[:, 0:4] + p[:, 4:8]
    q = q[:, 0:2] + q[:, 2:4]
    return q[:, 0:1] + q[:, 1:2]


def _codebook_sumsq_kernel(e_ref, o_ref):
    o_ref[...] = _rowsumsq(e_ref[...])


def _select_kernel(x_ref, kk_ref, e_ref, b_ref, idx_ref):
    x = x_ref[...]
    e = e_ref[...]
    b = b_ref[...]
    kk = kk_ref[...]
    c = jax.lax.dot_general(x, e, (((1,), (1,)), ((), ())),
                            preferred_element_type=jnp.float32)
    a = _rowsumsq(x)
    dist = (a + b) - 2.0 * c

    bits = jax.lax.bitcast_convert_type(dist, jnp.int32)
    key = jnp.where(bits >= 0, bits, (~bits) ^ jnp.int32(_INT_MIN))

    kk16 = kk.astype(jnp.int16)

    def count16(mask):
        m = mask.astype(jnp.int16)
        w = _K // 2
        while w >= 128:
            m = m[:, :w] + m[:, w:]
            w //= 2
        return jnp.sum(m.astype(jnp.int32), axis=1, keepdims=True,
                       ).astype(jnp.int16)

    def search16(keys16, rank16, nbits, signed=True):
        if signed:
            cnt_neg = count16(keys16 < jnp.int16(0))
            neg = rank16 < cnt_neg
            prefix0 = jnp.where(neg, jnp.int16(-32768), jnp.int16(0))
            cntp0 = jnp.where(neg, jnp.int16(0), cnt_neg)
        else:
            prefix0 = jnp.zeros_like(rank16)
            cntp0 = jnp.zeros_like(rank16)

        def body(t, carry):
            prefix, cntp = carry
            bit = (jnp.int32(1) << (jnp.int32(nbits - 1) - t)).astype(jnp.int16)
            cand = prefix | bit
            cnt = count16(keys16 < cand)
            take = cnt <= rank16
            return jnp.where(take, cand, prefix), jnp.where(take, cnt, cntp)

        return jax.lax.fori_loop(0, nbits, body, (prefix0, cntp0))

    khi = (key >> 16).astype(jnp.int16)
    vhi, cnt_hi = search16(khi, kk16, 15)

    eqhi = khi == vhi
    klo = jnp.where(eqhi, key.astype(jnp.int16) ^ jnp.int16(-32768),
                    jnp.int16(32767))
    vlo, cnt_lo = search16(klo, kk16 - cnt_hi, 15)

    iota = jax.lax.broadcasted_iota(jnp.int16, (_RB, _K), 1)
    tie_keys = jnp.where(eqhi & (klo == vlo), iota, jnp.int16(_INT16_BIG))
    sel16, _ = search16(tie_keys, kk16 - cnt_hi - cnt_lo, 13, signed=False)
    idx_ref[...] = sel16.astype(jnp.int32)


def _finalize_kernel(l_ref, q_ref, o_ref, loss_ref, acc_ref):
    i = pl.program_id(0)
    lat = l_ref[...]
    q = q_ref[...]
    diff = q - lat
    o_ref[...] = lat + diff

    @pl.when(i == 0)
    def _():
        acc_ref[0, 0] = 0.0

    acc_ref[0, 0] += jnp.sum(diff * diff)

    @pl.when(i == _NB - 1)
    def _():
        mse = acc_ref[0, 0] / (_N * _D)
        loss_ref[...] = jnp.full((1, 1), mse * _BETA + mse, jnp.float32)


def _sc_gather(table, idx):
    mesh = plsc.VectorSubcoreMesh(core_axis_name="c", subcore_axis_name="s")

    @functools.partial(
        pl.kernel, mesh=mesh,
        out_type=jax.ShapeDtypeStruct((_N, _D), jnp.float32),
        scratch_types=[
            pltpu.VMEM((_BPW,), jnp.int32),
            pltpu.VMEM((_BPW, _D), jnp.float32),
            pltpu.SemaphoreType.DMA,
        ],
    )
    def gather_k(table_hbm, idx_hbm, out_hbm, idx_v, rows_v, sem):
        wid = jax.lax.axis_index("s") * 2 + jax.lax.axis_index("c")
        base = wid * _BPW
        pltpu.sync_copy(idx_hbm.at[pl.ds(base, _BPW)], idx_v)
        pltpu.async_copy(table_hbm.at[idx_v], rows_v, sem).wait()
        pltpu.sync_copy(rows_v, out_hbm.at[pl.ds(base, _BPW)])

    return gather_k(table, idx)


def kernel(latents, k, embedding):
    flat = latents.reshape(_N, _D)
    kk = jnp.tile(k.astype(jnp.int32), _RB // _C).reshape(_RB, 1)

    bcol = pl.pallas_call(
        _codebook_sumsq_kernel,
        grid=(_K // 512,),
        in_specs=[pl.BlockSpec((512, _D), lambda i: (i, 0))],
        out_specs=pl.BlockSpec((512, 1), lambda i: (i, 0)),
        out_shape=jax.ShapeDtypeStruct((_K, 1), jnp.float32),
    )(embedding)
    brow = bcol.reshape(1, _K)

    inds = pl.pallas_call(
        _select_kernel,
        grid=(_NB,),
        in_specs=[pl.BlockSpec((_RB, _D), lambda i: (i, 0)),
                  pl.BlockSpec((_RB, 1), lambda i: (0, 0)),
                  pl.BlockSpec((_K, _D), lambda i: (0, 0)),
                  pl.BlockSpec((1, _K), lambda i: (0, 0))],
        out_specs=pl.BlockSpec((_RB, 1), lambda i: (i, 0)),
        out_shape=jax.ShapeDtypeStruct((_N, 1), jnp.int32),
    )(flat, kk, embedding, brow)

    quant = _sc_gather(embedding, inds.reshape(_N))

    quant_st, loss = pl.pallas_call(
        _finalize_kernel,
        grid=(_NB,),
        in_specs=[pl.BlockSpec((_RB, _D), lambda i: (i, 0)),
                  pl.BlockSpec((_RB, _D), lambda i: (i, 0))],
        out_specs=[pl.BlockSpec((_RB, _D), lambda i: (i, 0)),
                   pl.BlockSpec((1, 1), lambda i: (0, 0))],
        out_shape=[jax.ShapeDtypeStruct((_N, _D), jnp.float32),
                   jax.ShapeDtypeStruct((1, 1), jnp.float32)],
        scratch_shapes=[pltpu.SMEM((1, 1), jnp.float32)],
    )(flat, quant)

    return quant_st.reshape(latents.shape), loss.reshape(())


# speedup vs baseline: 1.2822x; 1.2822x over previous
import functools

import jax
import jax.numpy as jnp
from jax.experimental import pallas as pl
from jax.experimental.pallas import tpu as pltpu
from jax.experimental.pallas import tpu_sc as plsc

_D = 256
_K = 8192
_C = 32
_BETA = 0.25
_RB = 512
_N = 8 * 576
_NB = _N // _RB
_INT_MIN = -2147483648
_INT16_BIG = 8192

_NW = 32
_BPW = _N // _NW


def _rowsumsq(x):
    y = x * x
    s = y[:, :128] + y[:, 128:]
    p = s[:, 0:8] + s[:, 8:16]
    for g in range(2, 16):
        p = p + s[:, 8 * g:8 * g + 8]
    q = p[:, 0:4] + p[:, 4:8]
    q = q[:, 0:2] + q[:, 2:4]
    return q[:, 0:1] + q[:, 1:2]


def _codebook_sumsq_kernel(e_ref, o_ref):
    o_ref[...] = _rowsumsq(e_ref[...])


def _select_kernel(x_ref, kk_ref, e_ref, b_ref, idx_ref):
    x = x_ref[...]
    e = e_ref[...]
    b = b_ref[...]
    kk = kk_ref[...]
    c = jax.lax.dot_general(x, e, (((1,), (1,)), ((), ())),
                            preferred_element_type=jnp.float32)
    a = _rowsumsq(x)
    dist = (a + b) - 2.0 * c

    bits = jax.lax.bitcast_convert_type(dist, jnp.int32)
    key = jnp.where(bits >= 0, bits, (~bits) ^ jnp.int32(_INT_MIN))

    kk16 = kk.astype(jnp.int16)

    def count16(mask):
        m = mask.astype(jnp.int16)
        w = _K
        while w > 128:
            q = w // 4
            m = (m[:, :q] + m[:, q:2 * q]) + (m[:, 2 * q:3 * q] + m[:, 3 * q:])
            w = q
        return jnp.sum(m.astype(jnp.int32), axis=1, keepdims=True,
                       ).astype(jnp.int16)

    def search16(keys16, rank16, nbits):
        cnt_neg = count16(keys16 < jnp.int16(0))
        neg = rank16 < cnt_neg
        prefix0 = jnp.where(neg, jnp.int16(-32768), jnp.int16(0))
        cntp0 = jnp.where(neg, jnp.int16(0), cnt_neg)

        def body(t, carry):
            prefix, cntp = carry
            bit = (jnp.int32(1) << (jnp.int32(nbits - 1) - t)).astype(jnp.int16)
            cand = prefix | bit
            cnt = count16(keys16 < cand)
            take = cnt <= rank16
            return jnp.where(take, cand, prefix), jnp.where(take, cnt, cntp)

        return jax.lax.fori_loop(0, nbits, body, (prefix0, cntp0))

    khi = (key >> 16).astype(jnp.int16)
    vhi, cnt_hi = search16(khi, kk16, 15)

    eqhi = khi == vhi
    klo = jnp.where(eqhi, key.astype(jnp.int16) ^ jnp.int16(-32768),
                    jnp.int16(32767))
    vlo, cnt_lo = search16(klo, kk16 - cnt_hi, 15)

    iota = jax.lax.broadcasted_iota(jnp.int16, (_RB, _K), 1)
    tie_keys = jnp.where(eqhi & (klo == vlo), iota, jnp.int16(_INT16_BIG))
    sel16, _ = search16(tie_keys, kk16 - cnt_hi - cnt_lo, 13)
    idx_ref[...] = sel16.astype(jnp.int32)


def _finalize_kernel(l_ref, q_ref, o_ref, loss_ref, acc_ref):
    i = pl.program_id(0)
    lat = l_ref[...]
    q = q_ref[...]
    diff = q - lat
    o_ref[...] = lat + diff

    @pl.when(i == 0)
    def _():
        acc_ref[0, 0] = 0.0

    acc_ref[0, 0] += jnp.sum(diff * diff)

    @pl.when(i == _NB - 1)
    def _():
        mse = acc_ref[0, 0] / (_N * _D)
        loss_ref[...] = jnp.full((1, 1), mse * _BETA + mse, jnp.float32)


def _sc_gather(table, idx):
    mesh = plsc.VectorSubcoreMesh(core_axis_name="c", subcore_axis_name="s")

    @functools.partial(
        pl.kernel, mesh=mesh,
        out_type=jax.ShapeDtypeStruct((_N, _D), jnp.float32),
        scratch_types=[
            pltpu.VMEM((_BPW,), jnp.int32),
            pltpu.VMEM((_BPW, _D), jnp.float32),
            pltpu.SemaphoreType.DMA,
        ],
    )
    def gather_k(table_hbm, idx_hbm, out_hbm, idx_v, rows_v, sem):
        wid = jax.lax.axis_index("s") * 2 + jax.lax.axis_index("c")
        base = wid * _BPW
        pltpu.sync_copy(idx_hbm.at[pl.ds(base, _BPW)], idx_v)
        pltpu.async_copy(table_hbm.at[idx_v], rows_v, sem).wait()
        pltpu.sync_copy(rows_v, out_hbm.at[pl.ds(base, _BPW)])

    return gather_k(table, idx)


def kernel(latents, k, embedding):
    flat = latents.reshape(_N, _D)
    kk = jnp.tile(k.astype(jnp.int32), _RB // _C).reshape(_RB, 1)

    bcol = pl.pallas_call(
        _codebook_sumsq_kernel,
        grid=(_K // 512,),
        in_specs=[pl.BlockSpec((512, _D), lambda i: (i, 0))],
        out_specs=pl.BlockSpec((512, 1), lambda i: (i, 0)),
        out_shape=jax.ShapeDtypeStruct((_K, 1), jnp.float32),
    )(embedding)
    brow = bcol.reshape(1, _K)

    inds = pl.pallas_call(
        _select_kernel,
        grid=(_NB,),
        in_specs=[pl.BlockSpec((_RB, _D), lambda i: (i, 0)),
                  pl.BlockSpec((_RB, 1), lambda i: (0, 0)),
                  pl.BlockSpec((_K, _D), lambda i: (0, 0)),
                  pl.BlockSpec((1, _K), lambda i: (0, 0))],
        out_specs=pl.BlockSpec((_RB, 1), lambda i: (i, 0)),
        out_shape=jax.ShapeDtypeStruct((_N, 1), jnp.int32),
    )(flat, kk, embedding, brow)

    quant = _sc_gather(embedding, inds.reshape(_N))

    quant_st, loss = pl.pallas_call(
        _finalize_kernel,
        grid=(_NB,),
        in_specs=[pl.BlockSpec((_RB, _D), lambda i: (i, 0)),
                  pl.BlockSpec((_RB, _D), lambda i: (i, 0))],
        out_specs=[pl.BlockSpec((_RB, _D), lambda i: (i, 0)),
                   pl.BlockSpec((1, 1), lambda i: (0, 0))],
        out_shape=[jax.ShapeDtypeStruct((_N, _D), jnp.float32),
                   jax.ShapeDtypeStruct((1, 1), jnp.float32)],
        scratch_shapes=[pltpu.SMEM((1, 1), jnp.float32)],
    )(flat, quant)

    return quant_st.reshape(latents.shape), loss.reshape(())


# speedup vs baseline: 1.3048x; 1.0177x over previous
import functools

import jax
import jax.numpy as jnp
from jax.experimental import pallas as pl
from jax.experimental.pallas import tpu as pltpu
from jax.experimental.pallas import tpu_sc as plsc

_D = 256
_K = 8192
_C = 32
_BETA = 0.25
_RB = 512
_N = 8 * 576
_NB = _N // _RB
_INT_MIN = -2147483648
_INT16_BIG = 8192

_NW = 32
_BPW = _N // _NW


def _rowsumsq(x):
    y = x * x
    s = y[:, :128] + y[:, 128:]
    p = s[:, 0:8] + s[:, 8:16]
    for g in range(2, 16):
        p = p + s[:, 8 * g:8 * g + 8]
    q = p[:, 0:4] + p[:, 4:8]
    q = q[:, 0:2] + q[:, 2:4]
    return q[:, 0:1] + q[:, 1:2]


def _codebook_sumsq_kernel(e_ref, o_ref):
    o_ref[...] = _rowsumsq(e_ref[...])


def _select_kernel(x_ref, kk_ref, e_ref, b_ref, idx_ref):
    x = x_ref[...]
    e = e_ref[...]
    b = b_ref[...]
    kk = kk_ref[...]
    c = jax.lax.dot_general(x, e, (((1,), (1,)), ((), ())),
                            preferred_element_type=jnp.float32)
    a = _rowsumsq(x)
    dist = (a + b) - 2.0 * c

    bits = jax.lax.bitcast_convert_type(dist, jnp.int32)
    key = jnp.where(bits >= 0, bits, (~bits) ^ jnp.int32(_INT_MIN))

    kk16 = kk.astype(jnp.int16)

    def count16(mask):
        m = mask.astype(jnp.int16)
        w = _K // 2
        while w >= 128:
            m = m[:, :w] + m[:, w:]
            w //= 2
        return jnp.sum(m.astype(jnp.int32), axis=1, keepdims=True,
                       ).astype(jnp.int16)

    def search16(keys16, rank16, nbits):
        cnt_neg = count16(keys16 < jnp.int16(0))
        neg = rank16 < cnt_neg
        prefix0 = jnp.where(neg, jnp.int16(-32768), jnp.int16(0))
        cntp0 = jnp.where(neg, jnp.int16(0), cnt_neg)

        def body(t, carry):
            prefix, cntp = carry
            bit = (jnp.int32(1) << (jnp.int32(nbits - 1) - t)).astype(jnp.int16)
            cand = prefix | bit
            cnt = count16(keys16 < cand)
            take = cnt <= rank16
            return jnp.where(take, cand, prefix), jnp.where(take, cnt, cntp)

        return jax.lax.fori_loop(0, nbits, body, (prefix0, cntp0))

    khi = (key >> 16).astype(jnp.int16)
    vhi, cnt_hi = search16(khi, kk16, 15)

    eqhi = khi == vhi
    klo = jnp.where(eqhi, key.astype(jnp.int16) ^ jnp.int16(-32768),
                    jnp.int16(32767))
    vlo, cnt_lo = search16(klo, kk16 - cnt_hi, 15)

    iota = jax.lax.broadcasted_iota(jnp.int16, (_RB, _K), 1)
    tie_keys = jnp.where(eqhi & (klo == vlo), iota, jnp.int16(_INT16_BIG))
    sel16, _ = search16(tie_keys, kk16 - cnt_hi - cnt_lo, 13)
    idx_ref[...] = sel16.astype(jnp.int32)


def _finalize_kernel(l_ref, q_ref, o_ref, loss_ref, acc_ref):
    i = pl.program_id(0)
    lat = l_ref[...]
    q = q_ref[...]
    diff = q - lat
    o_ref[...] = lat + diff

    @pl.when(i == 0)
    def _():
        acc_ref[0, 0] = 0.0

    acc_ref[0, 0] += jnp.sum(diff * diff)

    @pl.when(i == _NB - 1)
    def _():
        mse = acc_ref[0, 0] / (_N * _D)
        loss_ref[...] = jnp.full((1, 1), mse * _BETA + mse, jnp.float32)


def _sc_gather(table, idx):
    mesh = plsc.VectorSubcoreMesh(core_axis_name="c", subcore_axis_name="s")

    @functools.partial(
        pl.kernel, mesh=mesh,
        out_type=jax.ShapeDtypeStruct((_N, _D), jnp.float32),
        scratch_types=[
            pltpu.VMEM((_BPW,), jnp.int32),
            pltpu.VMEM((_BPW, _D), jnp.float32),
            pltpu.SemaphoreType.DMA,
        ],
    )
    def gather_k(table_hbm, idx_hbm, out_hbm, idx_v, rows_v, sem):
        wid = jax.lax.axis_index("s") * 2 + jax.lax.axis_index("c")
        base = wid * _BPW
        pltpu.sync_copy(idx_hbm.at[pl.ds(base, _BPW)], idx_v)
        pltpu.async_copy(table_hbm.at[idx_v], rows_v, sem).wait()
        pltpu.sync_copy(rows_v, out_hbm.at[pl.ds(base, _BPW)])

    return gather_k(table, idx)


def kernel(latents, k, embedding):
    flat = latents.reshape(_N, _D)
    kk = jnp.tile(k.astype(jnp.int32), _RB // _C).reshape(_RB, 1)

    bcol = pl.pallas_call(
        _codebook_sumsq_kernel,
        grid=(_K // 512,),
        in_specs=[pl.BlockSpec((512, _D), lambda i: (i, 0))],
        out_specs=pl.BlockSpec((512, 1), lambda i: (i, 0)),
        out_shape=jax.ShapeDtypeStruct((_K, 1), jnp.float32),
    )(embedding)
    brow = bcol.reshape(1, _K)

    inds = pl.pallas_call(
        _select_kernel,
        grid=(_NB,),
        in_specs=[pl.BlockSpec((_RB, _D), lambda i: (i, 0)),
                  pl.BlockSpec((_RB, 1), lambda i: (0, 0)),
                  pl.BlockSpec((_K, _D), lambda i: (0, 0)),
                  pl.BlockSpec((1, _K), lambda i: (0, 0))],
        out_specs=pl.BlockSpec((_RB, 1), lambda i: (i, 0)),
        out_shape=jax.ShapeDtypeStruct((_N, 1), jnp.int32),
    )(flat, kk, embedding, brow)

    quant = _sc_gather(embedding, inds.reshape(_N))

    quant_st, loss = pl.pallas_call(
        _finalize_kernel,
        grid=(_NB,),
        in_specs=[pl.BlockSpec((_RB, _D), lambda i: (i, 0)),
                  pl.BlockSpec((_RB, _D), lambda i: (i, 0))],
        out_specs=[pl.BlockSpec((_RB, _D), lambda i: (i, 0)),
                   pl.BlockSpec((1, 1), lambda i: (0, 0))],
        out_shape=[jax.ShapeDtypeStruct((_N, _D), jnp.float32),
                   jax.ShapeDtypeStruct((1, 1), jnp.float32)],
        scratch_shapes=[pltpu.SMEM((1, 1), jnp.float32)],
    )(flat, quant)

    return quant_st.reshape(latents.shape), loss.reshape(())
